# baseline (speedup 1.0000x reference)
import functools

import jax
import jax.numpy as jnp
from jax import lax
from jax.experimental import pallas as pl
from jax.experimental.pallas import tpu as pltpu
from jax.experimental.pallas import tpu_sc as plsc

N = 10000
D = 128
E = 320000
H = 256

NC = 2
NS = 16
NW = NC * NS
CHUNK = 80
DEPTH = 3
IL = 2 * DEPTH
NCHUNK0 = 84
NCHUNK1 = 168
E0 = NS * NCHUNK0 * CHUNK
E_PAD = NS * (NCHUNK0 + NCHUNK1) * CHUNK
N_PAD = 10240
ROWS_PER_TILE = N_PAD // NS
ZCHUNK = 80
NZ = ROWS_PER_TILE // ZCHUNK


def _segsum_sc(x, src_r, dst_r):
    mesh = plsc.VectorSubcoreMesh(core_axis_name="c", subcore_axis_name="s")

    @functools.partial(
        pl.kernel,
        out_type=jax.ShapeDtypeStruct((NC, N_PAD, D), jnp.float32),
        mesh=mesh,
        scratch_types=[
            [pltpu.VMEM((CHUNK,), jnp.int32) for _ in range(IL)],
            [pltpu.VMEM((CHUNK,), jnp.int32) for _ in range(IL)],
            [pltpu.VMEM((CHUNK, D), jnp.float32) for _ in range(DEPTH)],
            pltpu.VMEM_SHARED((N_PAD, D), jnp.float32),
            [pltpu.SemaphoreType.DMA for _ in range(DEPTH)],
            [pltpu.SemaphoreType.DMA for _ in range(DEPTH)],
            pltpu.SemaphoreType.DMA,
        ],
    )
    def k(x_hbm, src_hbm, dst_hbm, out_hbm, sidx, didx, rows, acc, gsem, ssem, isem):
        stage_v = rows[0]
        c = lax.axis_index("c")
        s = lax.axis_index("s")
        row0 = s * ROWS_PER_TILE
        nchunk = lax.select(c == 0, NCHUNK0, NCHUNK1)
        ebase = lax.select(c == 0, s * (NCHUNK0 * CHUNK),
                           E0 + s * (NCHUNK1 * CHUNK))

        def zrow(i, t):
            def zlane(l, t2):
                stage_v[i, pl.ds(l * 16, 16)] = jnp.zeros((16,), jnp.float32)
                return t2
            return lax.fori_loop(0, D // 16, zlane, t)
        lax.fori_loop(0, ZCHUNK, zrow, 0)

        for j in range(NZ):
            pltpu.async_copy(stage_v.at[pl.ds(0, ZCHUNK)],
                             acc.at[pl.ds(row0 + j * ZCHUNK, ZCHUNK)], isem)
        for j in range(NZ):
            pltpu.make_async_copy(stage_v.at[pl.ds(0, ZCHUNK)],
                                  acc.at[pl.ds(row0 + j * ZCHUNK, ZCHUNK)], isem).wait()
        plsc.subcore_barrier()

        def idx_issue(j, il):
            pltpu.async_copy(src_hbm.at[pl.ds(ebase + j * CHUNK, CHUNK)], sidx[il], isem)
            pltpu.async_copy(dst_hbm.at[pl.ds(ebase + j * CHUNK, CHUNK)], didx[il], isem)

        def idx_wait(j, il):
            pltpu.make_async_copy(src_hbm.at[pl.ds(ebase + j * CHUNK, CHUNK)], sidx[il], isem).wait()
            pltpu.make_async_copy(dst_hbm.at[pl.ds(ebase + j * CHUNK, CHUNK)], didx[il], isem).wait()

        def scat_wait(b, il):
            pltpu.make_async_copy(rows[b], acc.at[didx[il]], ssem[b]).wait()

        for m in range(DEPTH):
            idx_issue(m, m)
            idx_wait(m, m)
            pltpu.async_copy(x_hbm.at[sidx[m]], rows[m], gsem[m])
        for m in range(DEPTH, IL):
            idx_issue(m, m)
        for m in range(DEPTH - 1):
            pltpu.make_async_copy(x_hbm.at[sidx[m]], rows[m], gsem[m]).wait()
            pltpu.async_copy(rows[m], acc.at[didx[m]], ssem[m], add=True)

        def step(j, il):
            b = il % DEPTH
            pb = (il + IL - 1) % IL
            scat_wait(b, (il + DEPTH) % IL)
            idx_wait(j, il)
            pltpu.async_copy(x_hbm.at[sidx[il]], rows[b], gsem[b])

            @pl.when(j + DEPTH < nchunk)
            def _():
                idx_issue(j + DEPTH, (il + DEPTH) % IL)

            pltpu.make_async_copy(x_hbm.at[sidx[pb]], rows[pb % DEPTH], gsem[pb % DEPTH]).wait()
            pltpu.async_copy(rows[pb % DEPTH], acc.at[didx[pb]], ssem[pb % DEPTH], add=True)

        def outerIL(q, t):
            for r in range(IL):
                j = IL * q + DEPTH + r

                @pl.when(j < nchunk)
                def _():
                    step(j, (DEPTH + r) % IL)
            return t
        lax.fori_loop(0, (nchunk - DEPTH + IL - 1) // IL, outerIL, 0)

        lastl = (NCHUNK0 - 1) % IL
        pltpu.make_async_copy(x_hbm.at[sidx[lastl]], rows[lastl % DEPTH],
                              gsem[lastl % DEPTH]).wait()
        pltpu.sync_copy(rows[lastl % DEPTH], acc.at[didx[lastl]], add=True)
        for dm in range(DEPTH, 1, -1):
            scat_wait((NCHUNK0 - dm) % DEPTH, (NCHUNK0 - dm) % IL)
        plsc.subcore_barrier()

        def rb_in(j, b):
            r = row0 + j * ZCHUNK
            pltpu.async_copy(acc.at[pl.ds(r, ZCHUNK)], rows[b], gsem[b])

        def rb_out(j, b):
            r = row0 + j * ZCHUNK
            pltpu.make_async_copy(acc.at[pl.ds(r, ZCHUNK)], rows[b], gsem[b]).wait()
            pltpu.async_copy(rows[b], out_hbm.at[c, pl.ds(r, ZCHUNK)], ssem[b])

        def rb_drain(j, b):
            r = row0 + j * ZCHUNK
            pltpu.make_async_copy(rows[b], out_hbm.at[c, pl.ds(r, ZCHUNK)], ssem[b]).wait()

        for j in range(min(DEPTH, NZ)):
            rb_in(j, j % DEPTH)
        for j in range(NZ):
            b = j % DEPTH
            rb_out(j, b)
            if j + DEPTH < NZ:
                rb_drain(j, b)
                rb_in(j + DEPTH, b)
        for j in range(max(0, NZ - DEPTH), NZ):
            rb_drain(j, j % DEPTH)

    return k(x, src_r, dst_r)


ROWBLK = 1000
GRID = N // ROWBLK

_rows_spec = pl.BlockSpec((ROWBLK, D), lambda i: (i, 0))
_out_spec = pl.BlockSpec((ROWBLK, D), lambda i: (i, 0))


def _full(shape):
    return pl.BlockSpec(shape, lambda i: tuple(0 for _ in shape))


def _mlp1_tc(hA, hB, x, W1a, b1a, W1b, b1b):
    def body(hA_r, hB_r, x_r, Wa_r, ba_r, Wb_r, bb_r, out_r):
        h = (hA_r[...] + hB_r[...]).astype(jnp.bfloat16)
        z = (jnp.dot(h, Wa_r[0:D, :], preferred_element_type=jnp.float32)
             + jnp.dot(x_r[...].astype(jnp.bfloat16), Wa_r[D:2 * D, :], preferred_element_type=jnp.float32)
             + ba_r[...])
        z = jnp.maximum(z, 0.0).astype(jnp.bfloat16)
        a = jnp.dot(z, Wb_r[...], preferred_element_type=jnp.float32) + bb_r[...]
        out_r[...] = jnp.maximum(a, 0.0)

    return pl.pallas_call(
        body,
        out_shape=jax.ShapeDtypeStruct((N, D), jnp.float32),
        grid=(GRID,),
        in_specs=[_rows_spec, _rows_spec, _rows_spec,
                  _full((2 * D, H)), _full((1, H)), _full((H, D)), _full((1, D))],
        out_specs=_out_spec,
    )(hA, hB, x, W1a, b1a.reshape(1, H), W1b, b1b.reshape(1, D))


def _mlp2_tc(hA, hB, a1, x, W2a, b2a, W2b, b2b):
    def body(hA_r, hB_r, a1_r, x_r, Wa_r, ba_r, Wb_r, bb_r, out_r):
        h = (hA_r[...] + hB_r[...]).astype(jnp.bfloat16)
        z = (jnp.dot(h, Wa_r[0:D, :], preferred_element_type=jnp.float32)
             + jnp.dot(a1_r[...].astype(jnp.bfloat16), Wa_r[D:2 * D, :], preferred_element_type=jnp.float32)
             + jnp.dot(x_r[...].astype(jnp.bfloat16), Wa_r[2 * D:3 * D, :], preferred_element_type=jnp.float32)
             + ba_r[...])
        z = jnp.maximum(z, 0.0).astype(jnp.bfloat16)
        a = jnp.dot(z, Wb_r[...], preferred_element_type=jnp.float32) + bb_r[...]
        out_r[...] = jnp.maximum(a, 0.0)

    return pl.pallas_call(
        body,
        out_shape=jax.ShapeDtypeStruct((N, D), jnp.float32),
        grid=(GRID,),
        in_specs=[_rows_spec, _rows_spec, _rows_spec, _rows_spec,
                  _full((3 * D, H)), _full((1, H)), _full((H, D)), _full((1, D))],
        out_specs=_out_spec,
    )(hA, hB, a1, x, W2a, b2a.reshape(1, H), W2b, b2b.reshape(1, D))


def _mlp3_tc(hA, hB, a2, x, W3a, b3a, W3b, b3b):
    def body(hA_r, hB_r, a2_r, x_r, Wa_r, ba_r, Wb_r, bb_r, out_r):
        h = (hA_r[...] + hB_r[...]).astype(jnp.bfloat16)
        z = (jnp.dot(h, Wa_r[0:D, :], preferred_element_type=jnp.float32)
             + jnp.dot(a2_r[...].astype(jnp.bfloat16), Wa_r[D:2 * D, :], preferred_element_type=jnp.float32)
             + jnp.dot(x_r[...].astype(jnp.bfloat16), Wa_r[2 * D:3 * D, :], preferred_element_type=jnp.float32)
             + ba_r[...])
        z = jnp.maximum(z, 0.0).astype(jnp.bfloat16)
        logits = jnp.dot(z, Wb_r[...], preferred_element_type=jnp.float32) + bb_r[...]
        m = jnp.max(logits, axis=1, keepdims=True)
        e = jnp.exp(logits - m)
        lse = jnp.log(jnp.sum(e, axis=1, keepdims=True))
        out_r[...] = logits - m - lse

    return pl.pallas_call(
        body,
        out_shape=jax.ShapeDtypeStruct((N, D), jnp.float32),
        grid=(GRID,),
        in_specs=[_rows_spec, _rows_spec, _rows_spec, _rows_spec,
                  _full((3 * D, H)), _full((1, H)), _full((H, D)), _full((1, D))],
        out_specs=_out_spec,
    )(hA, hB, a2, x, W3a, b3a.reshape(1, H), W3b, b3b.reshape(1, D))


def kernel(node_feature, edge_index, W1a, b1a, W1b, b1b,
           W2a, b2a, W2b, b2b, W3a, b3a, W3b, b3b):
    x = node_feature
    pad = E_PAD - E
    src = jnp.concatenate([edge_index[0], jnp.zeros((pad,), jnp.int32)])
    dst = jnp.concatenate([edge_index[1], jnp.full((pad,), N_PAD - 1, jnp.int32)])

    bf = jnp.bfloat16
    h1 = _segsum_sc(x, src, dst)
    a1 = _mlp1_tc(h1[0, :N], h1[1, :N], x, W1a.astype(bf), b1a, W1b.astype(bf), b1b)

    h2 = _segsum_sc(a1, src, dst)
    a2 = _mlp2_tc(h2[0, :N], h2[1, :N], a1, x, W2a.astype(bf), b2a, W2b.astype(bf), b2b)

    h3 = _segsum_sc(a2, src, dst)
    return _mlp3_tc(h3[0, :N], h3[1, :N], a2, x, W3a.astype(bf), b3a, W3b.astype(bf), b3b)


# speedup vs baseline: 1.1590x; 1.1590x over previous
import functools

import jax
import jax.numpy as jnp
from jax import lax
from jax.experimental import pallas as pl
from jax.experimental.pallas import tpu as pltpu
from jax.experimental.pallas import tpu_sc as plsc

N = 10000
D = 128
E = 320000
H = 256

NC = 2
NS = 16
NW = NC * NS
CHUNK = 80
DEPTH = 3
IL = 2 * DEPTH
NCHUNK0 = 168
NCHUNK1 = 84
E0 = NS * NCHUNK0 * CHUNK
E_PAD = NS * (NCHUNK0 + NCHUNK1) * CHUNK
N_PAD = 10240
ROWS_PER_TILE = N_PAD // NS
ZCHUNK = 80
NZ = ROWS_PER_TILE // ZCHUNK


def _segsum_sc(x, src_r, dst_r):
    mesh = plsc.VectorSubcoreMesh(core_axis_name="c", subcore_axis_name="s")

    @functools.partial(
        pl.kernel,
        out_type=jax.ShapeDtypeStruct((NC, N_PAD, D), jnp.float32),
        mesh=mesh,
        scratch_types=[
            [pltpu.VMEM((CHUNK,), jnp.int32) for _ in range(IL)],
            [pltpu.VMEM((CHUNK,), jnp.int32) for _ in range(IL)],
            [pltpu.VMEM((CHUNK, D), jnp.float32) for _ in range(DEPTH)],
            pltpu.VMEM_SHARED((N_PAD, D), jnp.float32),
            [pltpu.SemaphoreType.DMA for _ in range(DEPTH)],
            [pltpu.SemaphoreType.DMA for _ in range(DEPTH)],
            pltpu.SemaphoreType.DMA,
        ],
    )
    def k(x_hbm, src_hbm, dst_hbm, out_hbm, sidx, didx, rows, acc, gsem, ssem, isem):
        stage_v = rows[0]
        c = lax.axis_index("c")
        s = lax.axis_index("s")
        row0 = s * ROWS_PER_TILE
        nchunk = lax.select(c == 0, NCHUNK0, NCHUNK1)
        ebase = lax.select(c == 0, s * (NCHUNK0 * CHUNK),
                           E0 + s * (NCHUNK1 * CHUNK))

        def zrow(i, t):
            def zlane(l, t2):
                stage_v[i, pl.ds(l * 16, 16)] = jnp.zeros((16,), jnp.float32)
                return t2
            return lax.fori_loop(0, D // 16, zlane, t)
        lax.fori_loop(0, ZCHUNK, zrow, 0)

        for j in range(NZ):
            pltpu.async_copy(stage_v.at[pl.ds(0, ZCHUNK)],
                             acc.at[pl.ds(row0 + j * ZCHUNK, ZCHUNK)], isem)
        for j in range(NZ):
            pltpu.make_async_copy(stage_v.at[pl.ds(0, ZCHUNK)],
                                  acc.at[pl.ds(row0 + j * ZCHUNK, ZCHUNK)], isem).wait()
        plsc.subcore_barrier()

        def idx_issue(j, il):
            pltpu.async_copy(src_hbm.at[pl.ds(ebase + j * CHUNK, CHUNK)], sidx[il], isem)
            pltpu.async_copy(dst_hbm.at[pl.ds(ebase + j * CHUNK, CHUNK)], didx[il], isem)

        def idx_wait(j, il):
            pltpu.make_async_copy(src_hbm.at[pl.ds(ebase + j * CHUNK, CHUNK)], sidx[il], isem).wait()
            pltpu.make_async_copy(dst_hbm.at[pl.ds(ebase + j * CHUNK, CHUNK)], didx[il], isem).wait()

        def scat_wait(b, il):
            pltpu.make_async_copy(rows[b], acc.at[didx[il]], ssem[b]).wait()

        for m in range(DEPTH):
            idx_issue(m, m)
            idx_wait(m, m)
            pltpu.async_copy(x_hbm.at[sidx[m]], rows[m], gsem[m])
        for m in range(DEPTH, IL):
            idx_issue(m, m)
        for m in range(DEPTH - 1):
            pltpu.make_async_copy(x_hbm.at[sidx[m]], rows[m], gsem[m]).wait()
            pltpu.async_copy(rows[m], acc.at[didx[m]], ssem[m], add=True)

        def step(j, il):
            b = il % DEPTH
            pb = (il + IL - 1) % IL
            scat_wait(b, (il + DEPTH) % IL)
            idx_wait(j, il)
            pltpu.async_copy(x_hbm.at[sidx[il]], rows[b], gsem[b])

            @pl.when(j + DEPTH < nchunk)
            def _():
                idx_issue(j + DEPTH, (il + DEPTH) % IL)

            pltpu.make_async_copy(x_hbm.at[sidx[pb]], rows[pb % DEPTH], gsem[pb % DEPTH]).wait()
            pltpu.async_copy(rows[pb % DEPTH], acc.at[didx[pb]], ssem[pb % DEPTH], add=True)

        def outerIL(q, t):
            for r in range(IL):
                j = IL * q + DEPTH + r

                @pl.when(j < nchunk)
                def _():
                    step(j, (DEPTH + r) % IL)
            return t
        lax.fori_loop(0, (nchunk - DEPTH + IL - 1) // IL, outerIL, 0)

        lastl = (NCHUNK0 - 1) % IL
        pltpu.make_async_copy(x_hbm.at[sidx[lastl]], rows[lastl % DEPTH],
                              gsem[lastl % DEPTH]).wait()
        pltpu.sync_copy(rows[lastl % DEPTH], acc.at[didx[lastl]], add=True)
        for dm in range(DEPTH, 1, -1):
            scat_wait((NCHUNK0 - dm) % DEPTH, (NCHUNK0 - dm) % IL)
        plsc.subcore_barrier()

        def rb_in(j, b):
            r = row0 + j * ZCHUNK
            pltpu.async_copy(acc.at[pl.ds(r, ZCHUNK)], rows[b], gsem[b])

        def rb_out(j, b):
            r = row0 + j * ZCHUNK
            pltpu.make_async_copy(acc.at[pl.ds(r, ZCHUNK)], rows[b], gsem[b]).wait()
            pltpu.async_copy(rows[b], out_hbm.at[c, pl.ds(r, ZCHUNK)], ssem[b])

        def rb_drain(j, b):
            r = row0 + j * ZCHUNK
            pltpu.make_async_copy(rows[b], out_hbm.at[c, pl.ds(r, ZCHUNK)], ssem[b]).wait()

        for j in range(min(DEPTH, NZ)):
            rb_in(j, j % DEPTH)
        for j in range(NZ):
            b = j % DEPTH
            rb_out(j, b)
            if j + DEPTH < NZ:
                rb_drain(j, b)
                rb_in(j + DEPTH, b)
        for j in range(max(0, NZ - DEPTH), NZ):
            rb_drain(j, j % DEPTH)

    return k(x, src_r, dst_r)


ROWBLK = 1000
GRID = N // ROWBLK

_rows_spec = pl.BlockSpec((ROWBLK, D), lambda i: (i, 0))
_out_spec = pl.BlockSpec((ROWBLK, D), lambda i: (i, 0))


def _full(shape):
    return pl.BlockSpec(shape, lambda i: tuple(0 for _ in shape))


def _mlp1_tc(hA, hB, x, W1a, b1a, W1b, b1b):
    def body(hA_r, hB_r, x_r, Wa_r, ba_r, Wb_r, bb_r, out_r):
        h = (hA_r[...] + hB_r[...]).astype(jnp.bfloat16)
        z = (jnp.dot(h, Wa_r[0:D, :], preferred_element_type=jnp.float32)
             + jnp.dot(x_r[...].astype(jnp.bfloat16), Wa_r[D:2 * D, :], preferred_element_type=jnp.float32)
             + ba_r[...])
        z = jnp.maximum(z, 0.0).astype(jnp.bfloat16)
        a = jnp.dot(z, Wb_r[...], preferred_element_type=jnp.float32) + bb_r[...]
        out_r[...] = jnp.maximum(a, 0.0)

    return pl.pallas_call(
        body,
        out_shape=jax.ShapeDtypeStruct((N, D), jnp.float32),
        grid=(GRID,),
        in_specs=[_rows_spec, _rows_spec, _rows_spec,
                  _full((2 * D, H)), _full((1, H)), _full((H, D)), _full((1, D))],
        out_specs=_out_spec,
    )(hA, hB, x, W1a, b1a.reshape(1, H), W1b, b1b.reshape(1, D))


def _mlp2_tc(hA, hB, a1, x, W2a, b2a, W2b, b2b):
    def body(hA_r, hB_r, a1_r, x_r, Wa_r, ba_r, Wb_r, bb_r, out_r):
        h = (hA_r[...] + hB_r[...]).astype(jnp.bfloat16)
        z = (jnp.dot(h, Wa_r[0:D, :], preferred_element_type=jnp.float32)
             + jnp.dot(a1_r[...].astype(jnp.bfloat16), Wa_r[D:2 * D, :], preferred_element_type=jnp.float32)
             + jnp.dot(x_r[...].astype(jnp.bfloat16), Wa_r[2 * D:3 * D, :], preferred_element_type=jnp.float32)
             + ba_r[...])
        z = jnp.maximum(z, 0.0).astype(jnp.bfloat16)
        a = jnp.dot(z, Wb_r[...], preferred_element_type=jnp.float32) + bb_r[...]
        out_r[...] = jnp.maximum(a, 0.0)

    return pl.pallas_call(
        body,
        out_shape=jax.ShapeDtypeStruct((N, D), jnp.float32),
        grid=(GRID,),
        in_specs=[_rows_spec, _rows_spec, _rows_spec, _rows_spec,
                  _full((3 * D, H)), _full((1, H)), _full((H, D)), _full((1, D))],
        out_specs=_out_spec,
    )(hA, hB, a1, x, W2a, b2a.reshape(1, H), W2b, b2b.reshape(1, D))


def _mlp3_tc(hA, hB, a2, x, W3a, b3a, W3b, b3b):
    def body(hA_r, hB_r, a2_r, x_r, Wa_r, ba_r, Wb_r, bb_r, out_r):
        h = (hA_r[...] + hB_r[...]).astype(jnp.bfloat16)
        z = (jnp.dot(h, Wa_r[0:D, :], preferred_element_type=jnp.float32)
             + jnp.dot(a2_r[...].astype(jnp.bfloat16), Wa_r[D:2 * D, :], preferred_element_type=jnp.float32)
             + jnp.dot(x_r[...].astype(jnp.bfloat16), Wa_r[2 * D:3 * D, :], preferred_element_type=jnp.float32)
             + ba_r[...])
        z = jnp.maximum(z, 0.0).astype(jnp.bfloat16)
        logits = jnp.dot(z, Wb_r[...], preferred_element_type=jnp.float32) + bb_r[...]
        m = jnp.max(logits, axis=1, keepdims=True)
        e = jnp.exp(logits - m)
        lse = jnp.log(jnp.sum(e, axis=1, keepdims=True))
        out_r[...] = logits - m - lse

    return pl.pallas_call(
        body,
        out_shape=jax.ShapeDtypeStruct((N, D), jnp.float32),
        grid=(GRID,),
        in_specs=[_rows_spec, _rows_spec, _rows_spec, _rows_spec,
                  _full((3 * D, H)), _full((1, H)), _full((H, D)), _full((1, D))],
        out_specs=_out_spec,
    )(hA, hB, a2, x, W3a, b3a.reshape(1, H), W3b, b3b.reshape(1, D))


def kernel(node_feature, edge_index, W1a, b1a, W1b, b1b,
           W2a, b2a, W2b, b2b, W3a, b3a, W3b, b3b):
    x = node_feature
    pad = E_PAD - E
    src = jnp.concatenate([edge_index[0], jnp.zeros((pad,), jnp.int32)])
    dst = jnp.concatenate([edge_index[1], jnp.full((pad,), N_PAD - 1, jnp.int32)])

    bf = jnp.bfloat16
    h1 = _segsum_sc(x, src, dst)
    a1 = _mlp1_tc(h1[0, :N], h1[1, :N], x, W1a.astype(bf), b1a, W1b.astype(bf), b1b)

    h2 = _segsum_sc(a1, src, dst)
    a2 = _mlp2_tc(h2[0, :N], h2[1, :N], a1, x, W2a.astype(bf), b2a, W2b.astype(bf), b2b)

    h3 = _segsum_sc(a2, src, dst)
    return _mlp3_tc(h3[0, :N], h3[1, :N], a2, x, W3a.astype(bf), b3a, W3b.astype(bf), b3b)


# speedup vs baseline: 1.2040x; 1.0388x over previous
import functools

import jax
import jax.numpy as jnp
from jax import lax
from jax.experimental import pallas as pl
from jax.experimental.pallas import tpu as pltpu
from jax.experimental.pallas import tpu_sc as plsc

N = 10000
D = 128
E = 320000
H = 256

NC = 2
NS = 16
NW = NC * NS
CHUNK = 80
DEPTH = 3
IL = 2 * DEPTH
NCHUNK0 = 186
NCHUNK1 = 66
E0 = NS * NCHUNK0 * CHUNK
E_PAD = NS * (NCHUNK0 + NCHUNK1) * CHUNK
N_PAD = 10240
ROWS_PER_TILE = N_PAD // NS
ZCHUNK = 80
NZ = ROWS_PER_TILE // ZCHUNK


def _segsum_sc(x, src_r, dst_r):
    mesh = plsc.VectorSubcoreMesh(core_axis_name="c", subcore_axis_name="s")

    @functools.partial(
        pl.kernel,
        out_type=jax.ShapeDtypeStruct((NC, N_PAD, D), jnp.float32),
        mesh=mesh,
        scratch_types=[
            [pltpu.VMEM((CHUNK,), jnp.int32) for _ in range(IL)],
            [pltpu.VMEM((CHUNK,), jnp.int32) for _ in range(IL)],
            [pltpu.VMEM((CHUNK, D), jnp.float32) for _ in range(DEPTH)],
            pltpu.VMEM_SHARED((N_PAD, D), jnp.float32),
            [pltpu.SemaphoreType.DMA for _ in range(DEPTH)],
            [pltpu.SemaphoreType.DMA for _ in range(DEPTH)],
            pltpu.SemaphoreType.DMA,
        ],
    )
    def k(x_hbm, src_hbm, dst_hbm, out_hbm, sidx, didx, rows, acc, gsem, ssem, isem):
        stage_v = rows[0]
        c = lax.axis_index("c")
        s = lax.axis_index("s")
        row0 = s * ROWS_PER_TILE
        nchunk = lax.select(c == 0, NCHUNK0, NCHUNK1)
        ebase = lax.select(c == 0, s * (NCHUNK0 * CHUNK),
                           E0 + s * (NCHUNK1 * CHUNK))

        def zrow(i, t):
            def zlane(l, t2):
                stage_v[i, pl.ds(l * 16, 16)] = jnp.zeros((16,), jnp.float32)
                return t2
            return lax.fori_loop(0, D // 16, zlane, t)
        lax.fori_loop(0, ZCHUNK, zrow, 0)

        for j in range(NZ):
            pltpu.async_copy(stage_v.at[pl.ds(0, ZCHUNK)],
                             acc.at[pl.ds(row0 + j * ZCHUNK, ZCHUNK)], isem)
        for j in range(NZ):
            pltpu.make_async_copy(stage_v.at[pl.ds(0, ZCHUNK)],
                                  acc.at[pl.ds(row0 + j * ZCHUNK, ZCHUNK)], isem).wait()
        plsc.subcore_barrier()

        def idx_issue(j, il):
            pltpu.async_copy(src_hbm.at[pl.ds(ebase + j * CHUNK, CHUNK)], sidx[il], isem)
            pltpu.async_copy(dst_hbm.at[pl.ds(ebase + j * CHUNK, CHUNK)], didx[il], isem)

        def idx_wait(j, il):
            pltpu.make_async_copy(src_hbm.at[pl.ds(ebase + j * CHUNK, CHUNK)], sidx[il], isem).wait()
            pltpu.make_async_copy(dst_hbm.at[pl.ds(ebase + j * CHUNK, CHUNK)], didx[il], isem).wait()

        def scat_wait(b, il):
            pltpu.make_async_copy(rows[b], acc.at[didx[il]], ssem[b]).wait()

        for m in range(DEPTH):
            idx_issue(m, m)
            idx_wait(m, m)
            pltpu.async_copy(x_hbm.at[sidx[m]], rows[m], gsem[m])
        for m in range(DEPTH, IL):
            idx_issue(m, m)
        for m in range(DEPTH - 1):
            pltpu.make_async_copy(x_hbm.at[sidx[m]], rows[m], gsem[m]).wait()
            pltpu.async_copy(rows[m], acc.at[didx[m]], ssem[m], add=True)

        def step(j, il):
            b = il % DEPTH
            pb = (il + IL - 1) % IL
            scat_wait(b, (il + DEPTH) % IL)
            idx_wait(j, il)
            pltpu.async_copy(x_hbm.at[sidx[il]], rows[b], gsem[b])

            @pl.when(j + DEPTH < nchunk)
            def _():
                idx_issue(j + DEPTH, (il + DEPTH) % IL)

            pltpu.make_async_copy(x_hbm.at[sidx[pb]], rows[pb % DEPTH], gsem[pb % DEPTH]).wait()
            pltpu.async_copy(rows[pb % DEPTH], acc.at[didx[pb]], ssem[pb % DEPTH], add=True)

        def outerIL(q, t):
            for r in range(IL):
                j = IL * q + DEPTH + r

                @pl.when(j < nchunk)
                def _():
                    step(j, (DEPTH + r) % IL)
            return t
        lax.fori_loop(0, (nchunk - DEPTH + IL - 1) // IL, outerIL, 0)

        lastl = (NCHUNK0 - 1) % IL
        pltpu.make_async_copy(x_hbm.at[sidx[lastl]], rows[lastl % DEPTH],
                              gsem[lastl % DEPTH]).wait()
        pltpu.sync_copy(rows[lastl % DEPTH], acc.at[didx[lastl]], add=True)
        for dm in range(DEPTH, 1, -1):
            scat_wait((NCHUNK0 - dm) % DEPTH, (NCHUNK0 - dm) % IL)
        plsc.subcore_barrier()

        def rb_in(j, b):
            r = row0 + j * ZCHUNK
            pltpu.async_copy(acc.at[pl.ds(r, ZCHUNK)], rows[b], gsem[b])

        def rb_out(j, b):
            r = row0 + j * ZCHUNK
            pltpu.make_async_copy(acc.at[pl.ds(r, ZCHUNK)], rows[b], gsem[b]).wait()
            pltpu.async_copy(rows[b], out_hbm.at[c, pl.ds(r, ZCHUNK)], ssem[b])

        def rb_drain(j, b):
            r = row0 + j * ZCHUNK
            pltpu.make_async_copy(rows[b], out_hbm.at[c, pl.ds(r, ZCHUNK)], ssem[b]).wait()

        for j in range(min(DEPTH, NZ)):
            rb_in(j, j % DEPTH)
        for j in range(NZ):
            b = j % DEPTH
            rb_out(j, b)
            if j + DEPTH < NZ:
                rb_drain(j, b)
                rb_in(j + DEPTH, b)
        for j in range(max(0, NZ - DEPTH), NZ):
            rb_drain(j, j % DEPTH)

    return k(x, src_r, dst_r)


ROWBLK = 1000
GRID = N // ROWBLK

_rows_spec = pl.BlockSpec((ROWBLK, D), lambda i: (i, 0))
_out_spec = pl.BlockSpec((ROWBLK, D), lambda i: (i, 0))


def _full(shape):
    return pl.BlockSpec(shape, lambda i: tuple(0 for _ in shape))


def _mlp1_tc(hA, hB, x, W1a, b1a, W1b, b1b):
    def body(hA_r, hB_r, x_r, Wa_r, ba_r, Wb_r, bb_r, out_r):
        h = (hA_r[...] + hB_r[...]).astype(jnp.bfloat16)
        z = (jnp.dot(h, Wa_r[0:D, :], preferred_element_type=jnp.float32)
             + jnp.dot(x_r[...].astype(jnp.bfloat16), Wa_r[D:2 * D, :], preferred_element_type=jnp.float32)
             + ba_r[...])
        z = jnp.maximum(z, 0.0).astype(jnp.bfloat16)
        a = jnp.dot(z, Wb_r[...], preferred_element_type=jnp.float32) + bb_r[...]
        out_r[...] = jnp.maximum(a, 0.0)

    return pl.pallas_call(
        body,
        out_shape=jax.ShapeDtypeStruct((N, D), jnp.float32),
        grid=(GRID,),
        in_specs=[_rows_spec, _rows_spec, _rows_spec,
                  _full((2 * D, H)), _full((1, H)), _full((H, D)), _full((1, D))],
        out_specs=_out_spec,
    )(hA, hB, x, W1a, b1a.reshape(1, H), W1b, b1b.reshape(1, D))


def _mlp2_tc(hA, hB, a1, x, W2a, b2a, W2b, b2b):
    def body(hA_r, hB_r, a1_r, x_r, Wa_r, ba_r, Wb_r, bb_r, out_r):
        h = (hA_r[...] + hB_r[...]).astype(jnp.bfloat16)
        z = (jnp.dot(h, Wa_r[0:D, :], preferred_element_type=jnp.float32)
             + jnp.dot(a1_r[...].astype(jnp.bfloat16), Wa_r[D:2 * D, :], preferred_element_type=jnp.float32)
             + jnp.dot(x_r[...].astype(jnp.bfloat16), Wa_r[2 * D:3 * D, :], preferred_element_type=jnp.float32)
             + ba_r[...])
        z = jnp.maximum(z, 0.0).astype(jnp.bfloat16)
        a = jnp.dot(z, Wb_r[...], preferred_element_type=jnp.float32) + bb_r[...]
        out_r[...] = jnp.maximum(a, 0.0)

    return pl.pallas_call(
        body,
        out_shape=jax.ShapeDtypeStruct((N, D), jnp.float32),
        grid=(GRID,),
        in_specs=[_rows_spec, _rows_spec, _rows_spec, _rows_spec,
                  _full((3 * D, H)), _full((1, H)), _full((H, D)), _full((1, D))],
        out_specs=_out_spec,
    )(hA, hB, a1, x, W2a, b2a.reshape(1, H), W2b, b2b.reshape(1, D))


def _mlp3_tc(hA, hB, a2, x, W3a, b3a, W3b, b3b):
    def body(hA_r, hB_r, a2_r, x_r, Wa_r, ba_r, Wb_r, bb_r, out_r):
        h = (hA_r[...] + hB_r[...]).astype(jnp.bfloat16)
        z = (jnp.dot(h, Wa_r[0:D, :], preferred_element_type=jnp.float32)
             + jnp.dot(a2_r[...].astype(jnp.bfloat16), Wa_r[D:2 * D, :], preferred_element_type=jnp.float32)
             + jnp.dot(x_r[...].astype(jnp.bfloat16), Wa_r[2 * D:3 * D, :], preferred_element_type=jnp.float32)
             + ba_r[...])
        z = jnp.maximum(z, 0.0).astype(jnp.bfloat16)
        logits = jnp.dot(z, Wb_r[...], preferred_element_type=jnp.float32) + bb_r[...]
        m = jnp.max(logits, axis=1, keepdims=True)
        e = jnp.exp(logits - m)
        lse = jnp.log(jnp.sum(e, axis=1, keepdims=True))
        out_r[...] = logits - m - lse

    return pl.pallas_call(
        body,
        out_shape=jax.ShapeDtypeStruct((N, D), jnp.float32),
        grid=(GRID,),
        in_specs=[_rows_spec, _rows_spec, _rows_spec, _rows_spec,
                  _full((3 * D, H)), _full((1, H)), _full((H, D)), _full((1, D))],
        out_specs=_out_spec,
    )(hA, hB, a2, x, W3a, b3a.reshape(1, H), W3b, b3b.reshape(1, D))


def kernel(node_feature, edge_index, W1a, b1a, W1b, b1b,
           W2a, b2a, W2b, b2b, W3a, b3a, W3b, b3b):
    x = node_feature
    pad = E_PAD - E
    src = jnp.concatenate([edge_index[0], jnp.zeros((pad,), jnp.int32)])
    dst = jnp.concatenate([edge_index[1], jnp.full((pad,), N_PAD - 1, jnp.int32)])

    bf = jnp.bfloat16
    h1 = _segsum_sc(x, src, dst)
    a1 = _mlp1_tc(h1[0, :N], h1[1, :N], x, W1a.astype(bf), b1a, W1b.astype(bf), b1b)

    h2 = _segsum_sc(a1, src, dst)
    a2 = _mlp2_tc(h2[0, :N], h2[1, :N], a1, x, W2a.astype(bf), b2a, W2b.astype(bf), b2b)

    h3 = _segsum_sc(a2, src, dst)
    return _mlp3_tc(h3[0, :N], h3[1, :N], a2, x, W3a.astype(bf), b3a, W3b.astype(bf), b3b)


# speedup vs baseline: 1.2313x; 1.0227x over previous
import functools

import jax
import jax.numpy as jnp
from jax import lax
from jax.experimental import pallas as pl
from jax.experimental.pallas import tpu as pltpu
from jax.experimental.pallas import tpu_sc as plsc

N = 10000
D = 128
E = 320000
H = 256

NC = 2
NS = 16
NW = NC * NS
CHUNK = 80
DEPTH = 3
IL = 2 * DEPTH
NCHUNK0 = 198
NCHUNK1 = 54
E0 = NS * NCHUNK0 * CHUNK
E_PAD = NS * (NCHUNK0 + NCHUNK1) * CHUNK
N_PAD = 10240
ROWS_PER_TILE = N_PAD // NS
ZCHUNK = 80
NZ = ROWS_PER_TILE // ZCHUNK


def _segsum_sc(x, src_r, dst_r):
    mesh = plsc.VectorSubcoreMesh(core_axis_name="c", subcore_axis_name="s")

    @functools.partial(
        pl.kernel,
        out_type=jax.ShapeDtypeStruct((NC, N_PAD, D), jnp.float32),
        mesh=mesh,
        scratch_types=[
            [pltpu.VMEM((CHUNK,), jnp.int32) for _ in range(IL)],
            [pltpu.VMEM((CHUNK,), jnp.int32) for _ in range(IL)],
            [pltpu.VMEM((CHUNK, D), jnp.float32) for _ in range(DEPTH)],
            pltpu.VMEM_SHARED((N_PAD, D), jnp.float32),
            [pltpu.SemaphoreType.DMA for _ in range(DEPTH)],
            [pltpu.SemaphoreType.DMA for _ in range(DEPTH)],
            pltpu.SemaphoreType.DMA,
        ],
    )
    def k(x_hbm, src_hbm, dst_hbm, out_hbm, sidx, didx, rows, acc, gsem, ssem, isem):
        stage_v = rows[0]
        c = lax.axis_index("c")
        s = lax.axis_index("s")
        row0 = s * ROWS_PER_TILE
        nchunk = lax.select(c == 0, NCHUNK0, NCHUNK1)
        ebase = lax.select(c == 0, s * (NCHUNK0 * CHUNK),
                           E0 + s * (NCHUNK1 * CHUNK))

        def zrow(i, t):
            def zlane(l, t2):
                stage_v[i, pl.ds(l * 16, 16)] = jnp.zeros((16,), jnp.float32)
                return t2
            return lax.fori_loop(0, D // 16, zlane, t)
        lax.fori_loop(0, ZCHUNK, zrow, 0)

        for j in range(NZ):
            pltpu.async_copy(stage_v.at[pl.ds(0, ZCHUNK)],
                             acc.at[pl.ds(row0 + j * ZCHUNK, ZCHUNK)], isem)
        for j in range(NZ):
            pltpu.make_async_copy(stage_v.at[pl.ds(0, ZCHUNK)],
                                  acc.at[pl.ds(row0 + j * ZCHUNK, ZCHUNK)], isem).wait()
        plsc.subcore_barrier()

        def idx_issue(j, il):
            pltpu.async_copy(src_hbm.at[pl.ds(ebase + j * CHUNK, CHUNK)], sidx[il], isem)
            pltpu.async_copy(dst_hbm.at[pl.ds(ebase + j * CHUNK, CHUNK)], didx[il], isem)

        def idx_wait(j, il):
            pltpu.make_async_copy(src_hbm.at[pl.ds(ebase + j * CHUNK, CHUNK)], sidx[il], isem).wait()
            pltpu.make_async_copy(dst_hbm.at[pl.ds(ebase + j * CHUNK, CHUNK)], didx[il], isem).wait()

        def scat_wait(b, il):
            pltpu.make_async_copy(rows[b], acc.at[didx[il]], ssem[b]).wait()

        for m in range(DEPTH):
            idx_issue(m, m)
            idx_wait(m, m)
            pltpu.async_copy(x_hbm.at[sidx[m]], rows[m], gsem[m])
        for m in range(DEPTH, IL):
            idx_issue(m, m)
        for m in range(DEPTH - 1):
            pltpu.make_async_copy(x_hbm.at[sidx[m]], rows[m], gsem[m]).wait()
            pltpu.async_copy(rows[m], acc.at[didx[m]], ssem[m], add=True)

        def step(j, il):
            b = il % DEPTH
            pb = (il + IL - 1) % IL
            scat_wait(b, (il + DEPTH) % IL)
            idx_wait(j, il)
            pltpu.async_copy(x_hbm.at[sidx[il]], rows[b], gsem[b])

            @pl.when(j + DEPTH < nchunk)
            def _():
                idx_issue(j + DEPTH, (il + DEPTH) % IL)

            pltpu.make_async_copy(x_hbm.at[sidx[pb]], rows[pb % DEPTH], gsem[pb % DEPTH]).wait()
            pltpu.async_copy(rows[pb % DEPTH], acc.at[didx[pb]], ssem[pb % DEPTH], add=True)

        def outerIL(q, t):
            for r in range(IL):
                j = IL * q + DEPTH + r

                @pl.when(j < nchunk)
                def _():
                    step(j, (DEPTH + r) % IL)
            return t
        lax.fori_loop(0, (nchunk - DEPTH + IL - 1) // IL, outerIL, 0)

        lastl = (NCHUNK0 - 1) % IL
        pltpu.make_async_copy(x_hbm.at[sidx[lastl]], rows[lastl % DEPTH],
                              gsem[lastl % DEPTH]).wait()
        pltpu.sync_copy(rows[lastl % DEPTH], acc.at[didx[lastl]], add=True)
        for dm in range(DEPTH, 1, -1):
            scat_wait((NCHUNK0 - dm) % DEPTH, (NCHUNK0 - dm) % IL)
        plsc.subcore_barrier()

        def rb_in(j, b):
            r = row0 + j * ZCHUNK
            pltpu.async_copy(acc.at[pl.ds(r, ZCHUNK)], rows[b], gsem[b])

        def rb_out(j, b):
            r = row0 + j * ZCHUNK
            pltpu.make_async_copy(acc.at[pl.ds(r, ZCHUNK)], rows[b], gsem[b]).wait()
            pltpu.async_copy(rows[b], out_hbm.at[c, pl.ds(r, ZCHUNK)], ssem[b])

        def rb_drain(j, b):
            r = row0 + j * ZCHUNK
            pltpu.make_async_copy(rows[b], out_hbm.at[c, pl.ds(r, ZCHUNK)], ssem[b]).wait()

        for j in range(min(DEPTH, NZ)):
            rb_in(j, j % DEPTH)
        for j in range(NZ):
            b = j % DEPTH
            rb_out(j, b)
            if j + DEPTH < NZ:
                rb_drain(j, b)
                rb_in(j + DEPTH, b)
        for j in range(max(0, NZ - DEPTH), NZ):
            rb_drain(j, j % DEPTH)

    return k(x, src_r, dst_r)


ROWBLK = 1000
GRID = N // ROWBLK

_rows_spec = pl.BlockSpec((ROWBLK, D), lambda i: (i, 0))
_out_spec = pl.BlockSpec((ROWBLK, D), lambda i: (i, 0))


def _full(shape):
    return pl.BlockSpec(shape, lambda i: tuple(0 for _ in shape))


def _mlp1_tc(hA, hB, x, W1a, b1a, W1b, b1b):
    def body(hA_r, hB_r, x_r, Wa_r, ba_r, Wb_r, bb_r, out_r):
        h = (hA_r[...] + hB_r[...]).astype(jnp.bfloat16)
        z = (jnp.dot(h, Wa_r[0:D, :], preferred_element_type=jnp.float32)
             + jnp.dot(x_r[...].astype(jnp.bfloat16), Wa_r[D:2 * D, :], preferred_element_type=jnp.float32)
             + ba_r[...])
        z = jnp.maximum(z, 0.0).astype(jnp.bfloat16)
        a = jnp.dot(z, Wb_r[...], preferred_element_type=jnp.float32) + bb_r[...]
        out_r[...] = jnp.maximum(a, 0.0)

    return pl.pallas_call(
        body,
        out_shape=jax.ShapeDtypeStruct((N, D), jnp.float32),
        grid=(GRID,),
        in_specs=[_rows_spec, _rows_spec, _rows_spec,
                  _full((2 * D, H)), _full((1, H)), _full((H, D)), _full((1, D))],
        out_specs=_out_spec,
    )(hA, hB, x, W1a, b1a.reshape(1, H), W1b, b1b.reshape(1, D))


def _mlp2_tc(hA, hB, a1, x, W2a, b2a, W2b, b2b):
    def body(hA_r, hB_r, a1_r, x_r, Wa_r, ba_r, Wb_r, bb_r, out_r):
        h = (hA_r[...] + hB_r[...]).astype(jnp.bfloat16)
        z = (jnp.dot(h, Wa_r[0:D, :], preferred_element_type=jnp.float32)
             + jnp.dot(a1_r[...].astype(jnp.bfloat16), Wa_r[D:2 * D, :], preferred_element_type=jnp.float32)
             + jnp.dot(x_r[...].astype(jnp.bfloat16), Wa_r[2 * D:3 * D, :], preferred_element_type=jnp.float32)
             + ba_r[...])
        z = jnp.maximum(z, 0.0).astype(jnp.bfloat16)
        a = jnp.dot(z, Wb_r[...], preferred_element_type=jnp.float32) + bb_r[...]
        out_r[...] = jnp.maximum(a, 0.0)

    return pl.pallas_call(
        body,
        out_shape=jax.ShapeDtypeStruct((N, D), jnp.float32),
        grid=(GRID,),
        in_specs=[_rows_spec, _rows_spec, _rows_spec, _rows_spec,
                  _full((3 * D, H)), _full((1, H)), _full((H, D)), _full((1, D))],
        out_specs=_out_spec,
    )(hA, hB, a1, x, W2a, b2a.reshape(1, H), W2b, b2b.reshape(1, D))


def _mlp3_tc(hA, hB, a2, x, W3a, b3a, W3b, b3b):
    def body(hA_r, hB_r, a2_r, x_r, Wa_r, ba_r, Wb_r, bb_r, out_r):
        h = (hA_r[...] + hB_r[...]).astype(jnp.bfloat16)
        z = (jnp.dot(h, Wa_r[0:D, :], preferred_element_type=jnp.float32)
             + jnp.dot(a2_r[...].astype(jnp.bfloat16), Wa_r[D:2 * D, :], preferred_element_type=jnp.float32)
             + jnp.dot(x_r[...].astype(jnp.bfloat16), Wa_r[2 * D:3 * D, :], preferred_element_type=jnp.float32)
             + ba_r[...])
        z = jnp.maximum(z, 0.0).astype(jnp.bfloat16)
        logits = jnp.dot(z, Wb_r[...], preferred_element_type=jnp.float32) + bb_r[...]
        m = jnp.max(logits, axis=1, keepdims=True)
        e = jnp.exp(logits - m)
        lse = jnp.log(jnp.sum(e, axis=1, keepdims=True))
        out_r[...] = logits - m - lse

    return pl.pallas_call(
        body,
        out_shape=jax.ShapeDtypeStruct((N, D), jnp.float32),
        grid=(GRID,),
        in_specs=[_rows_spec, _rows_spec, _rows_spec, _rows_spec,
                  _full((3 * D, H)), _full((1, H)), _full((H, D)), _full((1, D))],
        out_specs=_out_spec,
    )(hA, hB, a2, x, W3a, b3a.reshape(1, H), W3b, b3b.reshape(1, D))


def kernel(node_feature, edge_index, W1a, b1a, W1b, b1b,
           W2a, b2a, W2b, b2b, W3a, b3a, W3b, b3b):
    x = node_feature
    pad = E_PAD - E
    src = jnp.concatenate([edge_index[0], jnp.zeros((pad,), jnp.int32)])
    dst = jnp.concatenate([edge_index[1], jnp.full((pad,), N_PAD - 1, jnp.int32)])

    bf = jnp.bfloat16
    h1 = _segsum_sc(x, src, dst)
    a1 = _mlp1_tc(h1[0, :N], h1[1, :N], x, W1a.astype(bf), b1a, W1b.astype(bf), b1b)

    h2 = _segsum_sc(a1, src, dst)
    a2 = _mlp2_tc(h2[0, :N], h2[1, :N], a1, x, W2a.astype(bf), b2a, W2b.astype(bf), b2b)

    h3 = _segsum_sc(a2, src, dst)
    return _mlp3_tc(h3[0, :N], h3[1, :N], a2, x, W3a.astype(bf), b3a, W3b.astype(bf), b3b)


# speedup vs baseline: 1.2381x; 1.0055x over previous
import functools

import jax
import jax.numpy as jnp
from jax import lax
from jax.experimental import pallas as pl
from jax.experimental.pallas import tpu as pltpu
from jax.experimental.pallas import tpu_sc as plsc

N = 10000
D = 128
E = 320000
H = 256

NC = 2
NS = 16
NW = NC * NS
CHUNK = 80
DEPTH = 3
IL = 2 * DEPTH
NCHUNK0 = 210
NCHUNK1 = 42
E0 = NS * NCHUNK0 * CHUNK
E_PAD = NS * (NCHUNK0 + NCHUNK1) * CHUNK
N_PAD = 10240
ROWS_PER_TILE = N_PAD // NS
ZCHUNK = 80
NZ = ROWS_PER_TILE // ZCHUNK


def _segsum_sc(x, src_r, dst_r):
    mesh = plsc.VectorSubcoreMesh(core_axis_name="c", subcore_axis_name="s")

    @functools.partial(
        pl.kernel,
        out_type=jax.ShapeDtypeStruct((NC, N_PAD, D), jnp.float32),
        mesh=mesh,
        scratch_types=[
            [pltpu.VMEM((CHUNK,), jnp.int32) for _ in range(IL)],
            [pltpu.VMEM((CHUNK,), jnp.int32) for _ in range(IL)],
            [pltpu.VMEM((CHUNK, D), jnp.float32) for _ in range(DEPTH)],
            pltpu.VMEM_SHARED((N_PAD, D), jnp.float32),
            [pltpu.SemaphoreType.DMA for _ in range(DEPTH)],
            [pltpu.SemaphoreType.DMA for _ in range(DEPTH)],
            pltpu.SemaphoreType.DMA,
        ],
    )
    def k(x_hbm, src_hbm, dst_hbm, out_hbm, sidx, didx, rows, acc, gsem, ssem, isem):
        stage_v = rows[0]
        c = lax.axis_index("c")
        s = lax.axis_index("s")
        row0 = s * ROWS_PER_TILE
        nchunk = lax.select(c == 0, NCHUNK0, NCHUNK1)
        ebase = lax.select(c == 0, s * (NCHUNK0 * CHUNK),
                           E0 + s * (NCHUNK1 * CHUNK))

        def zrow(i, t):
            def zlane(l, t2):
                stage_v[i, pl.ds(l * 16, 16)] = jnp.zeros((16,), jnp.float32)
                return t2
            return lax.fori_loop(0, D // 16, zlane, t)
        lax.fori_loop(0, ZCHUNK, zrow, 0)

        for j in range(NZ):
            pltpu.async_copy(stage_v.at[pl.ds(0, ZCHUNK)],
                             acc.at[pl.ds(row0 + j * ZCHUNK, ZCHUNK)], isem)
        for j in range(NZ):
            pltpu.make_async_copy(stage_v.at[pl.ds(0, ZCHUNK)],
                                  acc.at[pl.ds(row0 + j * ZCHUNK, ZCHUNK)], isem).wait()
        plsc.subcore_barrier()

        def idx_issue(j, il):
            pltpu.async_copy(src_hbm.at[pl.ds(ebase + j * CHUNK, CHUNK)], sidx[il], isem)
            pltpu.async_copy(dst_hbm.at[pl.ds(ebase + j * CHUNK, CHUNK)], didx[il], isem)

        def idx_wait(j, il):
            pltpu.make_async_copy(src_hbm.at[pl.ds(ebase + j * CHUNK, CHUNK)], sidx[il], isem).wait()
            pltpu.make_async_copy(dst_hbm.at[pl.ds(ebase + j * CHUNK, CHUNK)], didx[il], isem).wait()

        def scat_wait(b, il):
            pltpu.make_async_copy(rows[b], acc.at[didx[il]], ssem[b]).wait()

        for m in range(DEPTH):
            idx_issue(m, m)
            idx_wait(m, m)
            pltpu.async_copy(x_hbm.at[sidx[m]], rows[m], gsem[m])
        for m in range(DEPTH, IL):
            idx_issue(m, m)
        for m in range(DEPTH - 1):
            pltpu.make_async_copy(x_hbm.at[sidx[m]], rows[m], gsem[m]).wait()
            pltpu.async_copy(rows[m], acc.at[didx[m]], ssem[m], add=True)

        def step(j, il):
            b = il % DEPTH
            pb = (il + IL - 1) % IL
            scat_wait(b, (il + DEPTH) % IL)
            idx_wait(j, il)
            pltpu.async_copy(x_hbm.at[sidx[il]], rows[b], gsem[b])

            @pl.when(j + DEPTH < nchunk)
            def _():
                idx_issue(j + DEPTH, (il + DEPTH) % IL)

            pltpu.make_async_copy(x_hbm.at[sidx[pb]], rows[pb % DEPTH], gsem[pb % DEPTH]).wait()
            pltpu.async_copy(rows[pb % DEPTH], acc.at[didx[pb]], ssem[pb % DEPTH], add=True)

        def outerIL(q, t):
            for r in range(IL):
                j = IL * q + DEPTH + r

                @pl.when(j < nchunk)
                def _():
                    step(j, (DEPTH + r) % IL)
            return t
        lax.fori_loop(0, (nchunk - DEPTH + IL - 1) // IL, outerIL, 0)

        lastl = (NCHUNK0 - 1) % IL
        pltpu.make_async_copy(x_hbm.at[sidx[lastl]], rows[lastl % DEPTH],
                              gsem[lastl % DEPTH]).wait()
        pltpu.sync_copy(rows[lastl % DEPTH], acc.at[didx[lastl]], add=True)
        for dm in range(DEPTH, 1, -1):
            scat_wait((NCHUNK0 - dm) % DEPTH, (NCHUNK0 - dm) % IL)
        plsc.subcore_barrier()

        def rb_in(j, b):
            r = row0 + j * ZCHUNK
            pltpu.async_copy(acc.at[pl.ds(r, ZCHUNK)], rows[b], gsem[b])

        def rb_out(j, b):
            r = row0 + j * ZCHUNK
            pltpu.make_async_copy(acc.at[pl.ds(r, ZCHUNK)], rows[b], gsem[b]).wait()
            pltpu.async_copy(rows[b], out_hbm.at[c, pl.ds(r, ZCHUNK)], ssem[b])

        def rb_drain(j, b):
            r = row0 + j * ZCHUNK
            pltpu.make_async_copy(rows[b], out_hbm.at[c, pl.ds(r, ZCHUNK)], ssem[b]).wait()

        for j in range(min(DEPTH, NZ)):
            rb_in(j, j % DEPTH)
        for j in range(NZ):
            b = j % DEPTH
            rb_out(j, b)
            if j + DEPTH < NZ:
                rb_drain(j, b)
                rb_in(j + DEPTH, b)
        for j in range(max(0, NZ - DEPTH), NZ):
            rb_drain(j, j % DEPTH)

    return k(x, src_r, dst_r)


ROWBLK = 1000
GRID = N // ROWBLK

_rows_spec = pl.BlockSpec((ROWBLK, D), lambda i: (i, 0))
_out_spec = pl.BlockSpec((ROWBLK, D), lambda i: (i, 0))


def _full(shape):
    return pl.BlockSpec(shape, lambda i: tuple(0 for _ in shape))


def _mlp1_tc(hA, hB, x, W1a, b1a, W1b, b1b):
    def body(hA_r, hB_r, x_r, Wa_r, ba_r, Wb_r, bb_r, out_r):
        h = (hA_r[...] + hB_r[...]).astype(jnp.bfloat16)
        z = (jnp.dot(h, Wa_r[0:D, :], preferred_element_type=jnp.float32)
             + jnp.dot(x_r[...].astype(jnp.bfloat16), Wa_r[D:2 * D, :], preferred_element_type=jnp.float32)
             + ba_r[...])
        z = jnp.maximum(z, 0.0).astype(jnp.bfloat16)
        a = jnp.dot(z, Wb_r[...], preferred_element_type=jnp.float32) + bb_r[...]
        out_r[...] = jnp.maximum(a, 0.0)

    return pl.pallas_call(
        body,
        out_shape=jax.ShapeDtypeStruct((N, D), jnp.float32),
        grid=(GRID,),
        in_specs=[_rows_spec, _rows_spec, _rows_spec,
                  _full((2 * D, H)), _full((1, H)), _full((H, D)), _full((1, D))],
        out_specs=_out_spec,
    )(hA, hB, x, W1a, b1a.reshape(1, H), W1b, b1b.reshape(1, D))


def _mlp2_tc(hA, hB, a1, x, W2a, b2a, W2b, b2b):
    def body(hA_r, hB_r, a1_r, x_r, Wa_r, ba_r, Wb_r, bb_r, out_r):
        h = (hA_r[...] + hB_r[...]).astype(jnp.bfloat16)
        z = (jnp.dot(h, Wa_r[0:D, :], preferred_element_type=jnp.float32)
             + jnp.dot(a1_r[...].astype(jnp.bfloat16), Wa_r[D:2 * D, :], preferred_element_type=jnp.float32)
             + jnp.dot(x_r[...].astype(jnp.bfloat16), Wa_r[2 * D:3 * D, :], preferred_element_type=jnp.float32)
             + ba_r[...])
        z = jnp.maximum(z, 0.0).astype(jnp.bfloat16)
        a = jnp.dot(z, Wb_r[...], preferred_element_type=jnp.float32) + bb_r[...]
        out_r[...] = jnp.maximum(a, 0.0)

    return pl.pallas_call(
        body,
        out_shape=jax.ShapeDtypeStruct((N, D), jnp.float32),
        grid=(GRID,),
        in_specs=[_rows_spec, _rows_spec, _rows_spec, _rows_spec,
                  _full((3 * D, H)), _full((1, H)), _full((H, D)), _full((1, D))],
        out_specs=_out_spec,
    )(hA, hB, a1, x, W2a, b2a.reshape(1, H), W2b, b2b.reshape(1, D))


def _mlp3_tc(hA, hB, a2, x, W3a, b3a, W3b, b3b):
    def body(hA_r, hB_r, a2_r, x_r, Wa_r, ba_r, Wb_r, bb_r, out_r):
        h = (hA_r[...] + hB_r[...]).astype(jnp.bfloat16)
        z = (jnp.dot(h, Wa_r[0:D, :], preferred_element_type=jnp.float32)
             + jnp.dot(a2_r[...].astype(jnp.bfloat16), Wa_r[D:2 * D, :], preferred_element_type=jnp.float32)
             + jnp.dot(x_r[...].astype(jnp.bfloat16), Wa_r[2 * D:3 * D, :], preferred_element_type=jnp.float32)
             + ba_r[...])
        z = jnp.maximum(z, 0.0).astype(jnp.bfloat16)
        logits = jnp.dot(z, Wb_r[...], preferred_element_type=jnp.float32) + bb_r[...]
        m = jnp.max(logits, axis=1, keepdims=True)
        e = jnp.exp(logits - m)
        lse = jnp.log(jnp.sum(e, axis=1, keepdims=True))
        out_r[...] = logits - m - lse

    return pl.pallas_call(
        body,
        out_shape=jax.ShapeDtypeStruct((N, D), jnp.float32),
        grid=(GRID,),
        in_specs=[_rows_spec, _rows_spec, _rows_spec, _rows_spec,
                  _full((3 * D, H)), _full((1, H)), _full((H, D)), _full((1, D))],
        out_specs=_out_spec,
    )(hA, hB, a2, x, W3a, b3a.reshape(1, H), W3b, b3b.reshape(1, D))


def kernel(node_feature, edge_index, W1a, b1a, W1b, b1b,
           W2a, b2a, W2b, b2b, W3a, b3a, W3b, b3b):
    x = node_feature
    pad = E_PAD - E
    src = jnp.concatenate([edge_index[0], jnp.zeros((pad,), jnp.int32)])
    dst = jnp.concatenate([edge_index[1], jnp.full((pad,), N_PAD - 1, jnp.int32)])

    bf = jnp.bfloat16
    h1 = _segsum_sc(x, src, dst)
    a1 = _mlp1_tc(h1[0, :N], h1[1, :N], x, W1a.astype(bf), b1a, W1b.astype(bf), b1b)

    h2 = _segsum_sc(a1, src, dst)
    a2 = _mlp2_tc(h2[0, :N], h2[1, :N], a1, x, W2a.astype(bf), b2a, W2b.astype(bf), b2b)

    h3 = _segsum_sc(a2, src, dst)
    return _mlp3_tc(h3[0, :N], h3[1, :N], a2, x, W3a.astype(bf), b3a, W3b.astype(bf), b3b)
